# Initial kernel scaffold; baseline (speedup 1.0000x reference)
import jax
import jax.numpy as jnp
from jax.experimental import pallas as pl


def kernel(positions, adj, Ws, bs):
    raise NotImplementedError("write your pallas kernel here")


# speedup vs baseline: 1.3380x; 1.3380x over previous
import jax
import jax.numpy as jnp
from jax.experimental import pallas as pl

_TM = 400


def _elu(x):
    return jnp.where(x > 0.0, x, jnp.exp(x) - 1.0)


def _first_body(pos_ref, w_ref, oleft_ref, oright_ref):
    s = jnp.dot(pos_ref[...], w_ref[...], preferred_element_type=jnp.float32)
    sl = oleft_ref.shape[1]
    oleft_ref[...] = s[:, :sl].astype(jnp.bfloat16)
    oright_ref[...] = s[:, sl:]


def _mid_body(adj_ref, sleft_ref, sright_ref, bl_ref, br_ref, wtop_ref,
              wbot_ref, oleft_ref, oright_ref):
    agg = jnp.dot(adj_ref[...], sleft_ref[...],
                  preferred_element_type=jnp.float32)
    xl = _elu(agg + bl_ref[...])
    xr = _elu(sright_ref[...] + br_ref[...])
    s = (jnp.dot(xl, wtop_ref[...], preferred_element_type=jnp.float32)
         + jnp.dot(xr, wbot_ref[...], preferred_element_type=jnp.float32))
    sln = oleft_ref.shape[1]
    oleft_ref[...] = s[:, :sln].astype(jnp.bfloat16)
    oright_ref[...] = s[:, sln:]


def _last_body(adj_ref, sleft_ref, sright_ref, bl_ref, br_ref,
               oml_ref, omr_ref):
    agg = jnp.dot(adj_ref[...], sleft_ref[...],
                  preferred_element_type=jnp.float32)
    xl = _elu(agg + bl_ref[...])
    xr = _elu(sright_ref[...] + br_ref[...])
    ml = jnp.max(xl, axis=0, keepdims=True)
    mr = jnp.max(xr, axis=0, keepdims=True)
    i = pl.program_id(0)

    @pl.when(i == 0)
    def _():
        oml_ref[...] = ml
        omr_ref[...] = mr

    @pl.when(i != 0)
    def _():
        oml_ref[...] = jnp.maximum(oml_ref[...], ml)
        omr_ref[...] = jnp.maximum(omr_ref[...], mr)


def kernel(positions, adj, Ws, bs):
    n, fin0 = positions.shape
    adj_bf = adj.astype(jnp.bfloat16)
    fouts = [W.shape[1] for W in Ws]
    sls = [max(f // 3, 2) for f in fouts]
    nlayers = len(Ws)
    num_tiles = n // _TM

    f0, s0 = fouts[0], sls[0]
    sleft, sright = pl.pallas_call(
        _first_body,
        grid=(1,),
        in_specs=[
            pl.BlockSpec((n, fin0), lambda i: (0, 0)),
            pl.BlockSpec((fin0, f0), lambda i: (0, 0)),
        ],
        out_specs=[
            pl.BlockSpec((n, s0), lambda i: (0, 0)),
            pl.BlockSpec((n, f0 - s0), lambda i: (0, 0)),
        ],
        out_shape=[
            jax.ShapeDtypeStruct((n, s0), jnp.bfloat16),
            jax.ShapeDtypeStruct((n, f0 - s0), jnp.float32),
        ],
    )(positions, Ws[0])

    for L in range(nlayers - 1):
        sl, fout = sls[L], fouts[L]
        wr = fout - sl
        sln, fn = sls[L + 1], fouts[L + 1]
        b = bs[L].reshape(1, fout)
        bl, br = b[:, :sl], b[:, sl:]
        wtop, wbot = Ws[L + 1][:sl], Ws[L + 1][sl:]
        sleft, sright = pl.pallas_call(
            _mid_body,
            grid=(num_tiles,),
            in_specs=[
                pl.BlockSpec((_TM, n), lambda i: (i, 0)),
                pl.BlockSpec((n, sl), lambda i: (0, 0)),
                pl.BlockSpec((_TM, wr), lambda i: (i, 0)),
                pl.BlockSpec((1, sl), lambda i: (0, 0)),
                pl.BlockSpec((1, wr), lambda i: (0, 0)),
                pl.BlockSpec((sl, fn), lambda i: (0, 0)),
                pl.BlockSpec((wr, fn), lambda i: (0, 0)),
            ],
            out_specs=[
                pl.BlockSpec((_TM, sln), lambda i: (i, 0)),
                pl.BlockSpec((_TM, fn - sln), lambda i: (i, 0)),
            ],
            out_shape=[
                jax.ShapeDtypeStruct((n, sln), jnp.bfloat16),
                jax.ShapeDtypeStruct((n, fn - sln), jnp.float32),
            ],
        )(adj_bf, sleft, sright, bl, br, wtop, wbot)

    sl, fout = sls[-1], fouts[-1]
    wr = fout - sl
    b = bs[-1].reshape(1, fout)
    bl, br = b[:, :sl], b[:, sl:]
    ml, mr = pl.pallas_call(
        _last_body,
        grid=(num_tiles,),
        in_specs=[
            pl.BlockSpec((_TM, n), lambda i: (i, 0)),
            pl.BlockSpec((n, sl), lambda i: (0, 0)),
            pl.BlockSpec((_TM, wr), lambda i: (i, 0)),
            pl.BlockSpec((1, sl), lambda i: (0, 0)),
            pl.BlockSpec((1, wr), lambda i: (0, 0)),
        ],
        out_specs=[
            pl.BlockSpec((1, sl), lambda i: (0, 0)),
            pl.BlockSpec((1, wr), lambda i: (0, 0)),
        ],
        out_shape=[
            jax.ShapeDtypeStruct((1, sl), jnp.float32),
            jax.ShapeDtypeStruct((1, wr), jnp.float32),
        ],
    )(adj_bf, sleft, sright, bl, br)

    return jnp.concatenate([ml[0], mr[0]], axis=0)


# speedup vs baseline: 1.3888x; 1.0380x over previous
import jax
import jax.numpy as jnp
from jax.experimental import pallas as pl

_TM = 1000


def _elu(x):
    return jnp.where(x > 0.0, x, jnp.exp(x) - 1.0)


def _first_body(pos_ref, w_ref, oleft_ref, oright_ref):
    s = jnp.dot(pos_ref[...], w_ref[...], preferred_element_type=jnp.float32)
    sl = oleft_ref.shape[1]
    oleft_ref[...] = s[:, :sl].astype(jnp.bfloat16)
    oright_ref[...] = s[:, sl:]


def _mid_body(adj_ref, sleft_ref, sright_ref, bl_ref, br_ref, wtop_ref,
              wbot_ref, oleft_ref, oright_ref):
    agg = jnp.dot(adj_ref[...], sleft_ref[...],
                  preferred_element_type=jnp.float32)
    xl = _elu(agg + bl_ref[...])
    xr = _elu(sright_ref[...] + br_ref[...])
    s = (jnp.dot(xl, wtop_ref[...], preferred_element_type=jnp.float32)
         + jnp.dot(xr, wbot_ref[...], preferred_element_type=jnp.float32))
    sln = oleft_ref.shape[1]
    oleft_ref[...] = s[:, :sln].astype(jnp.bfloat16)
    oright_ref[...] = s[:, sln:]


def _last_body(adj_ref, sleft_ref, sright_ref, bl_ref, br_ref,
               oml_ref, omr_ref):
    agg = jnp.dot(adj_ref[...], sleft_ref[...],
                  preferred_element_type=jnp.float32)
    xl = _elu(agg + bl_ref[...])
    xr = _elu(sright_ref[...] + br_ref[...])
    ml = jnp.max(xl, axis=0, keepdims=True)
    mr = jnp.max(xr, axis=0, keepdims=True)
    i = pl.program_id(0)

    @pl.when(i == 0)
    def _():
        oml_ref[...] = ml
        omr_ref[...] = mr

    @pl.when(i != 0)
    def _():
        oml_ref[...] = jnp.maximum(oml_ref[...], ml)
        omr_ref[...] = jnp.maximum(omr_ref[...], mr)


def kernel(positions, adj, Ws, bs):
    n, fin0 = positions.shape
    adj_bf = adj.astype(jnp.bfloat16)
    fouts = [W.shape[1] for W in Ws]
    sls = [max(f // 3, 2) for f in fouts]
    nlayers = len(Ws)
    num_tiles = n // _TM

    f0, s0 = fouts[0], sls[0]
    sleft, sright = pl.pallas_call(
        _first_body,
        grid=(1,),
        in_specs=[
            pl.BlockSpec((n, fin0), lambda i: (0, 0)),
            pl.BlockSpec((fin0, f0), lambda i: (0, 0)),
        ],
        out_specs=[
            pl.BlockSpec((n, s0), lambda i: (0, 0)),
            pl.BlockSpec((n, f0 - s0), lambda i: (0, 0)),
        ],
        out_shape=[
            jax.ShapeDtypeStruct((n, s0), jnp.bfloat16),
            jax.ShapeDtypeStruct((n, f0 - s0), jnp.float32),
        ],
    )(positions, Ws[0])

    for L in range(nlayers - 1):
        sl, fout = sls[L], fouts[L]
        wr = fout - sl
        sln, fn = sls[L + 1], fouts[L + 1]
        b = bs[L].reshape(1, fout)
        bl, br = b[:, :sl], b[:, sl:]
        wtop, wbot = Ws[L + 1][:sl], Ws[L + 1][sl:]
        sleft, sright = pl.pallas_call(
            _mid_body,
            grid=(num_tiles,),
            in_specs=[
                pl.BlockSpec((_TM, n), lambda i: (i, 0)),
                pl.BlockSpec((n, sl), lambda i: (0, 0)),
                pl.BlockSpec((_TM, wr), lambda i: (i, 0)),
                pl.BlockSpec((1, sl), lambda i: (0, 0)),
                pl.BlockSpec((1, wr), lambda i: (0, 0)),
                pl.BlockSpec((sl, fn), lambda i: (0, 0)),
                pl.BlockSpec((wr, fn), lambda i: (0, 0)),
            ],
            out_specs=[
                pl.BlockSpec((_TM, sln), lambda i: (i, 0)),
                pl.BlockSpec((_TM, fn - sln), lambda i: (i, 0)),
            ],
            out_shape=[
                jax.ShapeDtypeStruct((n, sln), jnp.bfloat16),
                jax.ShapeDtypeStruct((n, fn - sln), jnp.float32),
            ],
        )(adj_bf, sleft, sright, bl, br, wtop, wbot)

    sl, fout = sls[-1], fouts[-1]
    wr = fout - sl
    b = bs[-1].reshape(1, fout)
    bl, br = b[:, :sl], b[:, sl:]
    ml, mr = pl.pallas_call(
        _last_body,
        grid=(num_tiles,),
        in_specs=[
            pl.BlockSpec((_TM, n), lambda i: (i, 0)),
            pl.BlockSpec((n, sl), lambda i: (0, 0)),
            pl.BlockSpec((_TM, wr), lambda i: (i, 0)),
            pl.BlockSpec((1, sl), lambda i: (0, 0)),
            pl.BlockSpec((1, wr), lambda i: (0, 0)),
        ],
        out_specs=[
            pl.BlockSpec((1, sl), lambda i: (0, 0)),
            pl.BlockSpec((1, wr), lambda i: (0, 0)),
        ],
        out_shape=[
            jax.ShapeDtypeStruct((1, sl), jnp.float32),
            jax.ShapeDtypeStruct((1, wr), jnp.float32),
        ],
    )(adj_bf, sleft, sright, bl, br)

    return jnp.concatenate([ml[0], mr[0]], axis=0)
